# baseline (speedup 1.0000x reference)
import functools

import jax
import jax.numpy as jnp
from jax import lax
from jax.experimental import pallas as pl
from jax.experimental.pallas import tpu as pltpu
from jax.experimental.pallas import tpu_sc as plsc

NC = 2
NS = 16
NW = NC * NS
LANES = 16

_SC_PARAMS = pltpu.CompilerParams(needs_layout_passes=False)


@functools.lru_cache(maxsize=None)
def _make_relayout(v, d):
    assert d == 64 and v % 2 == 0
    nfull = v // 128
    tail = v % 128
    assert tail in (0, 64)
    n_iter = (nfull + NW - 1) // NW
    mesh = plsc.VectorSubcoreMesh(core_axis_name="c", subcore_axis_name="s")

    @functools.partial(
        pl.kernel,
        mesh=mesh,
        out_type=jax.ShapeDtypeStruct((v // 2, 128), jnp.float32),
        scratch_types=[
            *([pltpu.VMEM((d, 128), jnp.float32)] * 2),
            *([pltpu.VMEM((d, 128), jnp.float32)] * 2),
            pltpu.VMEM((d, tail if tail else 64), jnp.float32),
            pltpu.VMEM((32, 128), jnp.float32),
            *([pltpu.SemaphoreType.DMA] * 2),
        ],
        compiler_params=_SC_PARAMS,
    )
    def relayout_kernel(tab_hbm, out_hbm, tin0, tin1, tout0, tout1,
                        ttin, ttout, sem0, sem1):
        tin = (tin0, tin1)
        tout = (tout0, tout1)
        sem = (sem0, sem1)
        wid = lax.axis_index("s") * NC + lax.axis_index("c")

        cols = [lax.iota(jnp.int32, LANES) + 16 * q for q in range(4)]

        def block_of(i):
            return i * NW + wid

        def load(i, slot):
            @pl.when(block_of(i) < nfull)
            def _():
                pltpu.async_copy(
                    tab_hbm.at[:, pl.ds(block_of(i) * 128, 128)],
                    tin[slot], sem[slot],
                )

        zeros = jnp.zeros((LANES,), jnp.int32)

        def transpose_block(src, dst, rows):
            def body(r, carry):
                for g in range(8):
                    idx1 = zeros + (2 * r + g // 4)
                    vv = plsc.load_gather(src, [cols[g % 4], idx1])
                    dst[r, pl.ds(16 * g, LANES)] = vv
                return carry
            lax.fori_loop(0, rows, body, 0)

        load(0, 0)
        load(1, 1)

        def outer(ii, carry):
            for s in range(2):
                i = ii * 2 + s
                k = i * NW + wid

                @pl.when(k < nfull)
                def _():
                    pltpu.make_async_copy(
                        tab_hbm.at[:, pl.ds(0, 128)], tin[s], sem[s]
                    ).wait()
                    transpose_block(tin[s], tout[s], d)
                    pltpu.sync_copy(tout[s], out_hbm.at[pl.ds(k * 64, 64)])
                    load(i + 2, s)
            return carry

        lax.fori_loop(0, (n_iter + 1) // 2, outer, 0)

        if tail:
            @pl.when(wid == NW - 1)
            def _():
                pltpu.sync_copy(tab_hbm.at[:, pl.ds(nfull * 128, tail)], ttin)

                def body(r, carry):
                    for g in range(8):
                        idx1 = zeros + (2 * r + g // 4)
                        vv = plsc.load_gather(ttin, [cols[g % 4], idx1])
                        ttout[r, pl.ds(16 * g, LANES)] = vv
                    return carry

                lax.fori_loop(0, tail // 2, body, 0)
                pltpu.sync_copy(
                    ttout, out_hbm.at[pl.ds(nfull * 64, tail // 2)]
                )

    return relayout_kernel


@functools.lru_cache(maxsize=None)
def _make_gather(b, h, v, d):
    assert d == 64 and h % 2 == 0 and b % 128 == 0
    n_units = (h // 2) * (b // 128)
    assert n_units % NW == 0
    n_iter = n_units // NW
    bc_per_h = b // 128
    mesh = plsc.VectorSubcoreMesh(core_axis_name="c", subcore_axis_name="s")

    @functools.partial(
        pl.kernel,
        mesh=mesh,
        out_type=jax.ShapeDtypeStruct((h * d, b), jnp.float32),
        scratch_types=[
            *([pltpu.VMEM((2, 128), jnp.int32)] * 4),
            *([pltpu.VMEM((2, 128), jnp.int32)] * 2),
            *([pltpu.VMEM((256, 128), jnp.float32)] * 2),
            pltpu.VMEM((d, 128), jnp.float32),
            *([pltpu.SemaphoreType.DMA] * 4),
            *([pltpu.SemaphoreType.DMA] * 2),
        ],
        compiler_params=_SC_PARAMS,
    )
    def gather_kernel(idx_hbm, tab_hbm, out_hbm,
                      ip0, ip1, ip2, ip3, pe0, pe1, dst0, dst1, ob,
                      is0, is1, is2, is3, gs0, gs1):
        ip = (ip0, ip1, ip2, ip3)
        isem = (is0, is1, is2, is3)
        pe = (pe0, pe1)
        dst = (dst0, dst1)
        gsem = (gs0, gs1)
        wid = lax.axis_index("s") * NC + lax.axis_index("c")
        iota = lax.iota(jnp.int32, LANES)

        def unit_of(i):
            return i * NW + wid

        def load_idx(i, islot):
            @pl.when(i < n_iter)
            def _():
                u = unit_of(i)
                hp = u // bc_per_h
                bc = u % bc_per_h
                pltpu.async_copy(
                    idx_hbm.at[pl.ds(2 * hp, 2), pl.ds(bc * 128, 128)],
                    ip[islot], isem[islot],
                )

        def fire_gathers(i, islot, slot):
            @pl.when(i < n_iter)
            def _():
                pltpu.make_async_copy(
                    idx_hbm.at[pl.ds(0, 2), pl.ds(0, 128)],
                    ip[islot], isem[islot],
                ).wait()
                for rh in range(2):
                    for g in range(8):
                        iv = ip[islot][rh, pl.ds(16 * g, LANES)]
                        pe[slot][rh, pl.ds(16 * g, LANES)] = (
                            lax.shift_left(jnp.bitwise_and(iv, 1), 6)
                        )
                        ip[islot][rh, pl.ds(16 * g, LANES)] = (
                            lax.shift_right_logical(iv, 1)
                        )
                for rh in range(2):
                    pltpu.async_copy(
                        tab_hbm.at[ip[islot].at[rh]],
                        dst[slot].at[pl.ds(rh * 128, 128)],
                        gsem[slot],
                    )

        for i in range(4):
            load_idx(i, i)
        fire_gathers(0, 0, 0)
        fire_gathers(1, 1, 1)

        def outer(ii, carry):
            for s in range(4):
                i = ii * 4 + s
                slot = s % 2

                @pl.when(i < n_iter)
                def _():
                    u = unit_of(i)
                    hp = u // bc_per_h
                    bc = u % bc_per_h
                    for rh in range(2):
                        pltpu.make_async_copy(
                            tab_hbm.at[ip[0].at[0]],
                            dst[slot].at[pl.ds(rh * 128, 128)],
                            gsem[slot],
                        ).wait()
                    for rh in range(2):
                        pvec = [
                            pe[slot][rh, pl.ds(16 * g, LANES)]
                            for g in range(8)
                        ]
                        rows = [iota + (rh * 128 + 16 * g) for g in range(8)]

                        def body(dd, carry2):
                            for g in range(8):
                                vv = plsc.load_gather(
                                    dst[slot], [rows[g], pvec[g] + dd]
                                )
                                ob[dd, pl.ds(16 * g, LANES)] = vv
                            return carry2

                        lax.fori_loop(0, d, body, 0)
                        pltpu.sync_copy(
                            ob,
                            out_hbm.at[
                                pl.ds((2 * hp + rh) * d, d),
                                pl.ds(bc * 128, 128),
                            ],
                        )
                    load_idx(i + 4, s)
                    fire_gathers(i + 2, (s + 2) % 4, slot)
            return carry

        lax.fori_loop(0, (n_iter + 3) // 4, outer, 0)

    return gather_kernel


def kernel(inp, table):
    b, h = inp.shape
    v, d = table.shape
    idx_t = inp.T.astype(jnp.int32)
    table_t = table.T
    tabp = _make_relayout(v, d)(table_t)
    out2 = _make_gather(b, h, v, d)(idx_t, tabp)
    return out2.reshape(h, d, b).transpose(2, 0, 1)


# speedup vs baseline: 2.2867x; 2.2867x over previous
import functools

import jax
import jax.numpy as jnp
from jax import lax
from jax.experimental import pallas as pl
from jax.experimental.pallas import tpu as pltpu
from jax.experimental.pallas import tpu_sc as plsc

NC = 2
NS = 16
NW = NC * NS
LANES = 16
CHUNK = 128
NBUF = 4
QUART = 4


@functools.lru_cache(maxsize=None)
def _make_gather(n, width):
    assert n % (CHUNK * NW) == 0 and width == 4 * LANES
    b_per_w = n // NW
    n_steps = b_per_w // CHUNK
    n_outer = (n_steps + NBUF - 1) // NBUF
    mesh = plsc.VectorSubcoreMesh(core_axis_name="c", subcore_axis_name="s")

    @functools.partial(
        pl.kernel,
        mesh=mesh,
        out_type=jax.ShapeDtypeStruct((n * QUART, LANES), jnp.float32),
        scratch_types=[
            pltpu.VMEM((n_steps, CHUNK), jnp.int32),
            *([pltpu.VMEM((QUART, CHUNK), jnp.int32)] * NBUF),
            *([pltpu.VMEM((QUART * CHUNK, LANES), jnp.float32)] * NBUF),
            *([pltpu.SemaphoreType.DMA] * NBUF),
        ],
        compiler_params=pltpu.CompilerParams(
            use_tc_tiling_on_sc=False, needs_layout_passes=False
        ),
    )
    def gather_kernel(idx_hbm, table_hbm, out_hbm, idx_v, *scratch):
        idx4 = scratch[:NBUF]
        rows = scratch[NBUF:2 * NBUF]
        gsem = scratch[2 * NBUF:]
        wid = lax.axis_index("s") * NC + lax.axis_index("c")
        base = wid * b_per_w
        iota4 = lax.iota(jnp.int32, LANES) * QUART

        pltpu.sync_copy(idx_hbm.at[pl.ds(wid * n_steps, n_steps)], idx_v)

        zero = jnp.zeros((LANES,), jnp.int32)

        def fire(j, slot):
            for g in range(8):
                iv = idx_v[j, pl.ds(LANES * g, LANES)]
                iv4 = lax.shift_left(iv, 2)
                row = zero + (g // 2)
                for q in range(QUART):
                    col = iota4 + (64 * (g % 2) + q)
                    plsc.store_scatter(idx4[slot], [row, col], iv4 + q)
            for q in range(QUART):
                pltpu.async_copy(
                    table_hbm.at[idx4[slot].at[q]],
                    rows[slot].at[pl.ds(q * CHUNK, CHUNK)],
                    gsem[slot],
                )

        for b in range(NBUF):
            fire(b, b)

        def outer(i, carry):
            for b in range(NBUF):
                j = i * NBUF + b
                jn = j + NBUF

                @pl.when(j < n_steps)
                def _():
                    for q in range(QUART):
                        pltpu.make_async_copy(
                            table_hbm.at[idx4[b].at[0]],
                            rows[b].at[pl.ds(q * CHUNK, CHUNK)],
                            gsem[b],
                        ).wait()
                    pltpu.sync_copy(
                        rows[b],
                        out_hbm.at[
                            pl.ds((base + j * CHUNK) * QUART, CHUNK * QUART)
                        ],
                    )

                    @pl.when(jn < n_steps)
                    def _():
                        fire(jn, b)

            return carry

        lax.fori_loop(0, n_outer, outer, 0)

    return gather_kernel


def kernel(inp, table):
    b, h = inp.shape
    v, width = table.shape
    idx = inp.reshape(b * h // CHUNK, CHUNK).astype(jnp.int32)
    tab16 = table.reshape(v * QUART, LANES)
    out = _make_gather(b * h, width)(idx, tab16)
    return out.reshape(b, h, width)


# speedup vs baseline: 2.2968x; 1.0044x over previous
import functools

import jax
import jax.numpy as jnp
from jax import lax
from jax.experimental import pallas as pl
from jax.experimental.pallas import tpu as pltpu
from jax.experimental.pallas import tpu_sc as plsc

NC = 2
NS = 16
NW = NC * NS
CHUNK = 256
NBUF = 4


@functools.lru_cache(maxsize=None)
def _make_gather(n, width):
    assert n % (CHUNK * NW) == 0
    b_per_w = n // NW
    n_steps = b_per_w // CHUNK
    n_outer = (n_steps + NBUF - 1) // NBUF
    mesh = plsc.VectorSubcoreMesh(core_axis_name="c", subcore_axis_name="s")

    @functools.partial(
        pl.kernel,
        mesh=mesh,
        out_type=jax.ShapeDtypeStruct((n, width), jnp.float32),
        scratch_types=[
            pltpu.VMEM((n_steps, CHUNK), jnp.int32),
            *([pltpu.VMEM((CHUNK, width), jnp.float32)] * NBUF),
            *([pltpu.SemaphoreType.DMA] * NBUF),
        ],
        compiler_params=pltpu.CompilerParams(use_tc_tiling_on_sc=False),
    )
    def gather_kernel(idx_hbm, table_hbm, out_hbm, idx_v, *bufs_and_sems):
        rows = bufs_and_sems[:NBUF]
        gsem = bufs_and_sems[NBUF:]
        wid = lax.axis_index("s") * NC + lax.axis_index("c")
        base = wid * b_per_w

        pltpu.sync_copy(idx_hbm.at[pl.ds(wid * n_steps, n_steps)], idx_v)

        for b in range(NBUF):
            pltpu.async_copy(table_hbm.at[idx_v.at[b]], rows[b], gsem[b])

        def outer(i, carry):
            for b in range(NBUF):
                j = i * NBUF + b
                jn = j + NBUF

                @pl.when(j < n_steps)
                def _():
                    pltpu.make_async_copy(
                        table_hbm.at[idx_v.at[0]], rows[b], gsem[b]
                    ).wait()
                    pltpu.sync_copy(
                        rows[b], out_hbm.at[pl.ds(base + j * CHUNK, CHUNK)]
                    )

                    @pl.when(jn < n_steps)
                    def _():
                        pltpu.async_copy(
                            table_hbm.at[idx_v.at[jn]], rows[b], gsem[b]
                        )

            return carry

        lax.fori_loop(0, n_outer, outer, 0)

    return gather_kernel


def kernel(inp, table):
    b, h = inp.shape
    _, width = table.shape
    idx = inp.reshape(b * h // CHUNK, CHUNK).astype(jnp.int32)
    out = _make_gather(b * h, width)(idx, table)
    return out.reshape(b, h, width)


# speedup vs baseline: 2.2993x; 1.0011x over previous
import functools

import jax
import jax.numpy as jnp
from jax import lax
from jax.experimental import pallas as pl
from jax.experimental.pallas import tpu as pltpu
from jax.experimental.pallas import tpu_sc as plsc

NC = 2
NS = 16
NW = NC * NS
CHUNK = 128
NBUF = 4


@functools.lru_cache(maxsize=None)
def _make_gather(n, width):
    assert n % (CHUNK * NW) == 0
    b_per_w = n // NW
    n_steps = b_per_w // CHUNK
    n_outer = (n_steps + NBUF - 1) // NBUF
    mesh = plsc.VectorSubcoreMesh(core_axis_name="c", subcore_axis_name="s")

    @functools.partial(
        pl.kernel,
        mesh=mesh,
        out_type=jax.ShapeDtypeStruct((n, width), jnp.float32),
        scratch_types=[
            pltpu.VMEM((n_steps, CHUNK), jnp.int32),
            *([pltpu.VMEM((CHUNK, width), jnp.float32)] * NBUF),
            *([pltpu.SemaphoreType.DMA] * NBUF),
        ],
        compiler_params=pltpu.CompilerParams(use_tc_tiling_on_sc=False),
    )
    def gather_kernel(idx_hbm, table_hbm, out_hbm, idx_v, *bufs_and_sems):
        rows = bufs_and_sems[:NBUF]
        gsem = bufs_and_sems[NBUF:]
        wid = lax.axis_index("s") * NC + lax.axis_index("c")
        base = wid * b_per_w

        pltpu.sync_copy(idx_hbm.at[pl.ds(wid * n_steps, n_steps)], idx_v)

        for b in range(NBUF):
            pltpu.async_copy(table_hbm.at[idx_v.at[b]], rows[b], gsem[b])

        def outer(i, carry):
            for b in range(NBUF):
                j = i * NBUF + b
                jn = j + NBUF

                @pl.when(j < n_steps)
                def _():
                    pltpu.make_async_copy(
                        table_hbm.at[idx_v.at[0]], rows[b], gsem[b]
                    ).wait()
                    pltpu.sync_copy(
                        rows[b], out_hbm.at[pl.ds(base + j * CHUNK, CHUNK)]
                    )

                    @pl.when(jn < n_steps)
                    def _():
                        pltpu.async_copy(
                            table_hbm.at[idx_v.at[jn]], rows[b], gsem[b]
                        )

            return carry

        lax.fori_loop(0, n_outer, outer, 0)

    return gather_kernel


def kernel(inp, table):
    b, h = inp.shape
    _, width = table.shape
    idx = inp.reshape(b * h // CHUNK, CHUNK).astype(jnp.int32)
    out = _make_gather(b * h, width)(idx, table)
    return out.reshape(b, h, width)
